# Initial kernel scaffold; baseline (speedup 1.0000x reference)
import jax
import jax.numpy as jnp
from jax.experimental import pallas as pl


def kernel(x, edge_index, W1, b1, W2, b2, Wc, bc):
    raise NotImplementedError("write your pallas kernel here")


# speedup vs baseline: 30.7190x; 30.7190x over previous
import functools

import jax
import jax.numpy as jnp
from jax import lax
from jax.experimental import pallas as pl
from jax.experimental.pallas import tpu as pltpu
from jax.experimental.pallas import tpu_sc as plsc

_N = 10000
_NP = 10240
_E = 320000
_D = 128
_H1 = 50
_HP1 = 64
_H2 = 2
_HP2 = 16
_C = 10
_DW = 8

_NC = 2
_NS = 16
_NW = _NC * _NS
_EPW = _E // _NW
_EB = 125
_EC = _EPW // _EB
_RPS = _NP // _NS
_ZB = 160
_ZC = _RPS // _ZB

_mesh = functools.partial(
    plsc.VectorSubcoreMesh, core_axis_name="c", subcore_axis_name="s"
)


def _deg_body(dst_hbm, ones_hbm, zrows_hbm, out_hbm, dst_v, ones_v, zrows_v,
              acc_sh, sem):
    cid = lax.axis_index("c")
    sid = lax.axis_index("s")
    wid = sid * _NC + cid
    base = sid * _RPS
    pltpu.sync_copy(zrows_hbm, zrows_v)
    for t in range(_ZC):
        pltpu.sync_copy(zrows_v, acc_sh.at[pl.ds(base + t * _ZB, _ZB)])
    pltpu.sync_copy(ones_hbm, ones_v)
    pltpu.sync_copy(dst_hbm.at[wid], dst_v)
    plsc.subcore_barrier()

    def step(j, carry):
        pltpu.sync_copy(ones_v, acc_sh.at[dst_v.at[j]], add=True)
        return carry

    lax.fori_loop(0, _EC, step, 0)
    plsc.subcore_barrier()
    pltpu.sync_copy(acc_sh.at[pl.ds(base, _RPS)],
                    out_hbm.at[cid, pl.ds(base, _RPS)])


_deg = pl.kernel(
    _deg_body,
    out_type=jax.ShapeDtypeStruct((_NC, _NP, _DW), jnp.float32),
    mesh=_mesh(),
    scratch_types=[
        pltpu.VMEM((_EC, _EB), jnp.int32),
        pltpu.VMEM((_EB, _DW), jnp.float32),
        pltpu.VMEM((_ZB, _DW), jnp.float32),
        pltpu.VMEM_SHARED((_NP, _DW), jnp.float32),
        pltpu.SemaphoreType.DMA,
    ],
    compiler_params=pltpu.CompilerParams(use_tc_tiling_on_sc=False),
)


def _make_agg(width):
    def body(y_hbm, src_hbm, dst_hbm, zrows_hbm, out_hbm,
             src_v, dst_v, rows_v, zrows_v, acc_sh, sem):
        cid = lax.axis_index("c")
        sid = lax.axis_index("s")
        wid = sid * _NC + cid
        base = sid * _RPS
        pltpu.sync_copy(zrows_hbm, zrows_v)
        for t in range(_ZC):
            pltpu.sync_copy(zrows_v, acc_sh.at[pl.ds(base + t * _ZB, _ZB)])
        pltpu.sync_copy(src_hbm.at[wid], src_v)
        pltpu.sync_copy(dst_hbm.at[wid], dst_v)
        plsc.subcore_barrier()

        def step(j, carry):
            pltpu.async_copy(y_hbm.at[src_v.at[j]], rows_v, sem).wait()
            pltpu.sync_copy(rows_v, acc_sh.at[dst_v.at[j]], add=True)
            return carry

        lax.fori_loop(0, _EC, step, 0)
        plsc.subcore_barrier()
        pltpu.sync_copy(acc_sh.at[pl.ds(base, _RPS)],
                        out_hbm.at[cid, pl.ds(base, _RPS)])

    return pl.kernel(
        body,
        out_type=jax.ShapeDtypeStruct((_NC, _NP, width), jnp.float32),
        mesh=_mesh(),
        scratch_types=[
            pltpu.VMEM((_EC, _EB), jnp.int32),
            pltpu.VMEM((_EC, _EB), jnp.int32),
            pltpu.VMEM((_EB, width), jnp.float32),
            pltpu.VMEM((_ZB, width), jnp.float32),
            pltpu.VMEM_SHARED((_NP, width), jnp.float32),
            pltpu.SemaphoreType.DMA,
        ],
        compiler_params=pltpu.CompilerParams(use_tc_tiling_on_sc=False),
    )


_agg1 = _make_agg(_HP1)
_agg2 = _make_agg(_HP2)

_R = 1000


def _tc_a_body(x_ref, w_ref, degp_ref, y1_ref, dinv_ref):
    deg = degp_ref[0] + degp_ref[1] + 1.0
    dinv = lax.rsqrt(jnp.maximum(deg, 1.0))
    dinv_ref[...] = dinv
    xw = jnp.dot(x_ref[...], w_ref[...], preferred_element_type=jnp.float32)
    y1_ref[...] = xw * dinv[:, 0:1]


_tc_a = pl.pallas_call(
    _tc_a_body,
    grid=(_N // _R,),
    in_specs=[
        pl.BlockSpec((_R, _D), lambda i: (i, 0)),
        pl.BlockSpec((_D, _HP1), lambda i: (0, 0)),
        pl.BlockSpec((_NC, _R, _DW), lambda i: (0, i, 0)),
    ],
    out_specs=[
        pl.BlockSpec((_R, _HP1), lambda i: (i, 0)),
        pl.BlockSpec((_R, _DW), lambda i: (i, 0)),
    ],
    out_shape=[
        jax.ShapeDtypeStruct((_N, _HP1), jnp.float32),
        jax.ShapeDtypeStruct((_N, _DW), jnp.float32),
    ],
)


def _tc_b_body(aggp_ref, y1_ref, dinv_ref, b1_ref, w2_ref, y2_ref):
    dinv = dinv_ref[:, 0:1]
    s = (aggp_ref[0] + aggp_ref[1] + y1_ref[...]) * dinv + b1_ref[...]
    h1 = jnp.tanh(s)
    y2_ref[...] = (
        jnp.dot(h1, w2_ref[...], preferred_element_type=jnp.float32) * dinv
    )


_tc_b = pl.pallas_call(
    _tc_b_body,
    grid=(_N // _R,),
    in_specs=[
        pl.BlockSpec((_NC, _R, _HP1), lambda i: (0, i, 0)),
        pl.BlockSpec((_R, _HP1), lambda i: (i, 0)),
        pl.BlockSpec((_R, _DW), lambda i: (i, 0)),
        pl.BlockSpec((1, _HP1), lambda i: (0, 0)),
        pl.BlockSpec((_HP1, _HP2), lambda i: (0, 0)),
    ],
    out_specs=pl.BlockSpec((_R, _HP2), lambda i: (i, 0)),
    out_shape=jax.ShapeDtypeStruct((_N, _HP2), jnp.float32),
)


def _tc_c_body(aggp_ref, y2_ref, dinv_ref, b2_ref, wc_ref, bc_ref,
               h_ref, out_ref):
    dinv = dinv_ref[:, 0:1]
    s = (aggp_ref[0] + aggp_ref[1] + y2_ref[...]) * dinv + b2_ref[...]
    h = jnp.tanh(s)
    h_ref[...] = h
    out_ref[...] = (
        jnp.dot(h, wc_ref[...], preferred_element_type=jnp.float32)
        + bc_ref[...]
    )


_tc_c = pl.pallas_call(
    _tc_c_body,
    grid=(_N // _R,),
    in_specs=[
        pl.BlockSpec((_NC, _R, _HP2), lambda i: (0, i, 0)),
        pl.BlockSpec((_R, _HP2), lambda i: (i, 0)),
        pl.BlockSpec((_R, _DW), lambda i: (i, 0)),
        pl.BlockSpec((1, _HP2), lambda i: (0, 0)),
        pl.BlockSpec((_HP2, 128), lambda i: (0, 0)),
        pl.BlockSpec((1, 128), lambda i: (0, 0)),
    ],
    out_specs=[
        pl.BlockSpec((_R, _HP2), lambda i: (i, 0)),
        pl.BlockSpec((_R, 128), lambda i: (i, 0)),
    ],
    out_shape=[
        jax.ShapeDtypeStruct((_N, _HP2), jnp.float32),
        jax.ShapeDtypeStruct((_N, 128), jnp.float32),
    ],
)


def kernel(x, edge_index, W1, b1, W2, b2, Wc, bc):
    src = edge_index[0].reshape(_NW, _EC, _EB)
    dst = edge_index[1].reshape(_NW, _EC, _EB)

    ones8 = jnp.ones((_EB, _DW), jnp.float32)
    z8 = jnp.zeros((_ZB, _DW), jnp.float32)
    zrows1 = jnp.zeros((_ZB, _HP1), jnp.float32)
    zrows2 = jnp.zeros((_ZB, _HP2), jnp.float32)

    W1p = jnp.pad(W1, ((0, 0), (0, _HP1 - _H1)))
    b1p = jnp.pad(b1, (0, _HP1 - _H1)).reshape(1, _HP1)
    W2p = jnp.pad(W2, ((0, _HP1 - _H1), (0, _HP2 - _H2)))
    b2p = jnp.pad(b2, (0, _HP2 - _H2)).reshape(1, _HP2)
    Wcp = jnp.pad(Wc, ((0, _HP2 - _H2), (0, 128 - _C)))
    bcp = jnp.pad(bc, (0, 128 - _C)).reshape(1, 128)

    degp = _deg(dst, ones8, z8)
    y1, dinv8 = _tc_a(x, W1p, degp)
    agg1 = _agg1(y1, src, dst, zrows1)
    y2 = _tc_b(agg1, y1, dinv8, b1p, W2p)
    agg2 = _agg2(y2, src, dst, zrows2)
    hpad, outpad = _tc_c(agg2, y2, dinv8, b2p, Wcp, bcp)
    return (outpad[:, :_C], hpad[:, :_H2])


# speedup vs baseline: 47.9230x; 1.5600x over previous
import functools

import jax
import jax.numpy as jnp
from jax import lax
from jax.experimental import pallas as pl
from jax.experimental.pallas import tpu as pltpu
from jax.experimental.pallas import tpu_sc as plsc

_N = 10000
_NP = 10240
_E = 320000
_D = 128
_H1 = 50
_HP1 = 64
_H2 = 2
_HP2 = 16
_C = 10
_DW = 8

_NC = 2
_NS = 16
_NW = _NC * _NS
_EPW = _E // _NW
_EB = 125
_EC = _EPW // _EB
_RPS = _NP // _NS
_ZB = 64
_ZC = _RPS // _ZB
_NB = 8
_NG = _EC // _NB

_mesh = functools.partial(
    plsc.VectorSubcoreMesh, core_axis_name="c", subcore_axis_name="s"
)


def _deg_body(dst_hbm, ones_hbm, zrows_hbm, out_hbm, dst_v, ones_v, zrows_v,
              acc_sh, sem):
    cid = lax.axis_index("c")
    sid = lax.axis_index("s")
    wid = sid * _NC + cid
    base = sid * _RPS
    pltpu.sync_copy(zrows_hbm, zrows_v)
    for t in range(_ZC):
        pltpu.sync_copy(zrows_v, acc_sh.at[pl.ds(base + t * _ZB, _ZB)])
    pltpu.sync_copy(ones_hbm, ones_v)
    pltpu.sync_copy(dst_hbm.at[wid], dst_v)
    plsc.subcore_barrier()

    def group(g, carry):
        descs = []
        for b in range(_NB):
            descs.append(
                pltpu.async_copy(ones_v, acc_sh.at[dst_v.at[g * _NB + b]],
                                 sem.at[b], add=True))
        for d in descs:
            d.wait()
        return carry

    lax.fori_loop(0, _NG, group, 0)
    plsc.subcore_barrier()
    pltpu.sync_copy(acc_sh.at[pl.ds(base, _RPS)],
                    out_hbm.at[cid, pl.ds(base, _RPS)])


_deg = pl.kernel(
    _deg_body,
    out_type=jax.ShapeDtypeStruct((_NC, _NP, _DW), jnp.float32),
    mesh=_mesh(),
    scratch_types=[
        pltpu.VMEM((_EC, _EB), jnp.int32),
        pltpu.VMEM((_EB, _DW), jnp.float32),
        pltpu.VMEM((_ZB, _DW), jnp.float32),
        pltpu.VMEM_SHARED((_NP, _DW), jnp.float32),
        pltpu.SemaphoreType.DMA((_NB,)),
    ],
    compiler_params=pltpu.CompilerParams(use_tc_tiling_on_sc=False),
)


def _make_agg(width):
    def body(y_hbm, src_hbm, dst_hbm, zrows_hbm, out_hbm,
             src_v, dst_v, rows_v, zrows_v, acc_sh, sem_g, sem_s):
        cid = lax.axis_index("c")
        sid = lax.axis_index("s")
        wid = sid * _NC + cid
        base = sid * _RPS
        pltpu.sync_copy(zrows_hbm, zrows_v)
        for t in range(_ZC):
            pltpu.sync_copy(zrows_v, acc_sh.at[pl.ds(base + t * _ZB, _ZB)])
        pltpu.sync_copy(src_hbm.at[wid], src_v)
        pltpu.sync_copy(dst_hbm.at[wid], dst_v)
        plsc.subcore_barrier()

        def group(g, carry):
            gds = []
            for b in range(_NB):
                j = g * _NB + b

                @pl.when(g > 0)
                def _wait_prev_scatter():
                    pltpu.make_async_copy(
                        rows_v.at[b], acc_sh.at[dst_v.at[j - _NB]],
                        sem_s.at[b]).wait()

                gds.append(
                    pltpu.async_copy(y_hbm.at[src_v.at[j]], rows_v.at[b],
                                     sem_g.at[b]))
            for b in range(_NB):
                j = g * _NB + b
                gds[b].wait()
                pltpu.async_copy(rows_v.at[b], acc_sh.at[dst_v.at[j]],
                                 sem_s.at[b], add=True)
            return carry

        lax.fori_loop(0, _NG, group, 0)
        for b in range(_NB):
            j = (_NG - 1) * _NB + b
            pltpu.make_async_copy(
                rows_v.at[b], acc_sh.at[dst_v.at[j]], sem_s.at[b]).wait()
        plsc.subcore_barrier()
        pltpu.sync_copy(acc_sh.at[pl.ds(base, _RPS)],
                        out_hbm.at[cid, pl.ds(base, _RPS)])

    return pl.kernel(
        body,
        out_type=jax.ShapeDtypeStruct((_NC, _NP, width), jnp.float32),
        mesh=_mesh(),
        scratch_types=[
            pltpu.VMEM((_EC, _EB), jnp.int32),
            pltpu.VMEM((_EC, _EB), jnp.int32),
            pltpu.VMEM((_NB, _EB, width), jnp.float32),
            pltpu.VMEM((_ZB, width), jnp.float32),
            pltpu.VMEM_SHARED((_NP, width), jnp.float32),
            pltpu.SemaphoreType.DMA((_NB,)),
            pltpu.SemaphoreType.DMA((_NB,)),
        ],
        compiler_params=pltpu.CompilerParams(use_tc_tiling_on_sc=False),
    )


_agg1 = _make_agg(_HP1)
_agg2 = _make_agg(_HP2)

_R = 1000


def _tc_a_body(x_ref, w_ref, degp_ref, y1_ref, dinv_ref):
    deg = degp_ref[0] + degp_ref[1] + 1.0
    dinv = lax.rsqrt(jnp.maximum(deg, 1.0))
    dinv_ref[...] = dinv
    xw = jnp.dot(x_ref[...], w_ref[...], preferred_element_type=jnp.float32)
    y1_ref[...] = xw * dinv[:, 0:1]


_tc_a = pl.pallas_call(
    _tc_a_body,
    grid=(_N // _R,),
    in_specs=[
        pl.BlockSpec((_R, _D), lambda i: (i, 0)),
        pl.BlockSpec((_D, _HP1), lambda i: (0, 0)),
        pl.BlockSpec((_NC, _R, _DW), lambda i: (0, i, 0)),
    ],
    out_specs=[
        pl.BlockSpec((_R, _HP1), lambda i: (i, 0)),
        pl.BlockSpec((_R, _DW), lambda i: (i, 0)),
    ],
    out_shape=[
        jax.ShapeDtypeStruct((_N, _HP1), jnp.float32),
        jax.ShapeDtypeStruct((_N, _DW), jnp.float32),
    ],
)


def _tc_b_body(aggp_ref, y1_ref, dinv_ref, b1_ref, w2_ref, y2_ref):
    dinv = dinv_ref[:, 0:1]
    s = (aggp_ref[0] + aggp_ref[1] + y1_ref[...]) * dinv + b1_ref[...]
    h1 = jnp.tanh(s)
    y2_ref[...] = (
        jnp.dot(h1, w2_ref[...], preferred_element_type=jnp.float32) * dinv
    )


_tc_b = pl.pallas_call(
    _tc_b_body,
    grid=(_N // _R,),
    in_specs=[
        pl.BlockSpec((_NC, _R, _HP1), lambda i: (0, i, 0)),
        pl.BlockSpec((_R, _HP1), lambda i: (i, 0)),
        pl.BlockSpec((_R, _DW), lambda i: (i, 0)),
        pl.BlockSpec((1, _HP1), lambda i: (0, 0)),
        pl.BlockSpec((_HP1, _HP2), lambda i: (0, 0)),
    ],
    out_specs=pl.BlockSpec((_R, _HP2), lambda i: (i, 0)),
    out_shape=jax.ShapeDtypeStruct((_N, _HP2), jnp.float32),
)


def _tc_c_body(aggp_ref, y2_ref, dinv_ref, b2_ref, wc_ref, bc_ref,
               h_ref, out_ref):
    dinv = dinv_ref[:, 0:1]
    s = (aggp_ref[0] + aggp_ref[1] + y2_ref[...]) * dinv + b2_ref[...]
    h = jnp.tanh(s)
    h_ref[...] = h
    out_ref[...] = (
        jnp.dot(h, wc_ref[...], preferred_element_type=jnp.float32)
        + bc_ref[...]
    )


_tc_c = pl.pallas_call(
    _tc_c_body,
    grid=(_N // _R,),
    in_specs=[
        pl.BlockSpec((_NC, _R, _HP2), lambda i: (0, i, 0)),
        pl.BlockSpec((_R, _HP2), lambda i: (i, 0)),
        pl.BlockSpec((_R, _DW), lambda i: (i, 0)),
        pl.BlockSpec((1, _HP2), lambda i: (0, 0)),
        pl.BlockSpec((_HP2, 128), lambda i: (0, 0)),
        pl.BlockSpec((1, 128), lambda i: (0, 0)),
    ],
    out_specs=[
        pl.BlockSpec((_R, _HP2), lambda i: (i, 0)),
        pl.BlockSpec((_R, 128), lambda i: (i, 0)),
    ],
    out_shape=[
        jax.ShapeDtypeStruct((_N, _HP2), jnp.float32),
        jax.ShapeDtypeStruct((_N, 128), jnp.float32),
    ],
)


def kernel(x, edge_index, W1, b1, W2, b2, Wc, bc):
    src = edge_index[0].reshape(_NW, _EC, _EB)
    dst = edge_index[1].reshape(_NW, _EC, _EB)

    ones8 = jnp.ones((_EB, _DW), jnp.float32)
    z8 = jnp.zeros((_ZB, _DW), jnp.float32)
    zrows1 = jnp.zeros((_ZB, _HP1), jnp.float32)
    zrows2 = jnp.zeros((_ZB, _HP2), jnp.float32)

    W1p = jnp.pad(W1, ((0, 0), (0, _HP1 - _H1)))
    b1p = jnp.pad(b1, (0, _HP1 - _H1)).reshape(1, _HP1)
    W2p = jnp.pad(W2, ((0, _HP1 - _H1), (0, _HP2 - _H2)))
    b2p = jnp.pad(b2, (0, _HP2 - _H2)).reshape(1, _HP2)
    Wcp = jnp.pad(Wc, ((0, _HP2 - _H2), (0, 128 - _C)))
    bcp = jnp.pad(bc, (0, 128 - _C)).reshape(1, 128)

    degp = _deg(dst, ones8, z8)
    y1, dinv8 = _tc_a(x, W1p, degp)
    agg1 = _agg1(y1, src, dst, zrows1)
    y2 = _tc_b(agg1, y1, dinv8, b1p, W2p)
    agg2 = _agg2(y2, src, dst, zrows2)
    hpad, outpad = _tc_c(agg2, y2, dinv8, b2p, Wcp, bcp)
    return (outpad[:, :_C], hpad[:, :_H2])


# speedup vs baseline: 48.1863x; 1.0055x over previous
import functools

import jax
import jax.numpy as jnp
from jax import lax
from jax.experimental import pallas as pl
from jax.experimental.pallas import tpu as pltpu
from jax.experimental.pallas import tpu_sc as plsc

_N = 10000
_NP = 10240
_E = 320000
_D = 128
_H1 = 50
_HP1 = 64
_H2 = 2
_HP2 = 16
_C = 10
_DW = 8

_NC = 2
_NS = 16
_NW = _NC * _NS
_EP = 327680
_EB = 128
_EC = 80
_RPS = _NP // _NS
_ZB = 32
_ZC = _RPS // _ZB
_NB = 8
_NG = _EC // _NB

_mesh = functools.partial(
    plsc.VectorSubcoreMesh, core_axis_name="c", subcore_axis_name="s"
)


def _deg_body(dst_hbm, ones_hbm, zrows_hbm, out_hbm, dst_v, ones_v, zrows_v,
              acc_sh, sem):
    cid = lax.axis_index("c")
    sid = lax.axis_index("s")
    wid = sid * _NC + cid
    base = sid * _RPS
    pltpu.sync_copy(zrows_hbm, zrows_v)
    for t in range(_ZC):
        pltpu.sync_copy(zrows_v, acc_sh.at[pl.ds(base + t * _ZB, _ZB)])
    pltpu.sync_copy(ones_hbm, ones_v)
    pltpu.sync_copy(dst_hbm.at[wid], dst_v)
    plsc.subcore_barrier()

    def group(g, carry):
        descs = []
        for b in range(_NB):
            descs.append(
                pltpu.async_copy(ones_v, acc_sh.at[dst_v.at[g * _NB + b]],
                                 sem.at[b], add=True))
        for d in descs:
            d.wait()
        return carry

    lax.fori_loop(0, _NG, group, 0)
    plsc.subcore_barrier()
    pltpu.sync_copy(acc_sh.at[pl.ds(base, _RPS)],
                    out_hbm.at[cid, pl.ds(base, _RPS)])


_deg = pl.kernel(
    _deg_body,
    out_type=jax.ShapeDtypeStruct((_NC, _NP, _DW), jnp.float32),
    mesh=_mesh(),
    scratch_types=[
        pltpu.VMEM((_EC, _EB), jnp.int32),
        pltpu.VMEM((_EB, _DW), jnp.float32),
        pltpu.VMEM((_ZB, _DW), jnp.float32),
        pltpu.VMEM_SHARED((_NP, _DW), jnp.float32),
        pltpu.SemaphoreType.DMA((_NB,)),
    ],
    compiler_params=pltpu.CompilerParams(use_tc_tiling_on_sc=False),
)


def _make_agg(width):
    def body(y_hbm, src_hbm, dst_hbm, zrows_hbm, out_hbm,
             src_v, dst_v, rows_v, zrows_v, acc_sh, sem_g, sem_s):
        cid = lax.axis_index("c")
        sid = lax.axis_index("s")
        wid = sid * _NC + cid
        base = sid * _RPS
        pltpu.sync_copy(zrows_hbm, zrows_v)
        for t in range(_ZC):
            pltpu.sync_copy(zrows_v, acc_sh.at[pl.ds(base + t * _ZB, _ZB)])
        pltpu.sync_copy(src_hbm.at[wid], src_v)
        pltpu.sync_copy(dst_hbm.at[wid], dst_v)
        plsc.subcore_barrier()

        def group(g, carry):
            gds = []
            for b in range(_NB):
                j = g * _NB + b

                @pl.when(g > 0)
                def _wait_prev_scatter():
                    pltpu.make_async_copy(
                        rows_v.at[b], acc_sh.at[dst_v.at[j - _NB]],
                        sem_s.at[b]).wait()

                gds.append(
                    pltpu.async_copy(y_hbm.at[src_v.at[j]], rows_v.at[b],
                                     sem_g.at[b]))
            for b in range(_NB):
                j = g * _NB + b
                gds[b].wait()
                pltpu.async_copy(rows_v.at[b], acc_sh.at[dst_v.at[j]],
                                 sem_s.at[b], add=True)
            return carry

        lax.fori_loop(0, _NG, group, 0)
        for b in range(_NB):
            j = (_NG - 1) * _NB + b
            pltpu.make_async_copy(
                rows_v.at[b], acc_sh.at[dst_v.at[j]], sem_s.at[b]).wait()
        plsc.subcore_barrier()
        pltpu.sync_copy(acc_sh.at[pl.ds(base, _RPS)],
                        out_hbm.at[cid, pl.ds(base, _RPS)])

    return pl.kernel(
        body,
        out_type=jax.ShapeDtypeStruct((_NC, _NP, width), jnp.float32),
        mesh=_mesh(),
        scratch_types=[
            pltpu.VMEM((_EC, _EB), jnp.int32),
            pltpu.VMEM((_EC, _EB), jnp.int32),
            pltpu.VMEM((_NB, _EB, width), jnp.float32),
            pltpu.VMEM((_ZB, width), jnp.float32),
            pltpu.VMEM_SHARED((_NP, width), jnp.float32),
            pltpu.SemaphoreType.DMA((_NB,)),
            pltpu.SemaphoreType.DMA((_NB,)),
        ],
        compiler_params=pltpu.CompilerParams(use_tc_tiling_on_sc=False),
    )


_agg1 = _make_agg(_HP1)
_agg2 = _make_agg(_HP2)

_R = 1000


def _tc_a_body(x_ref, w_ref, degp_ref, y1_ref, dinv_ref):
    deg = degp_ref[0] + degp_ref[1] + 1.0
    dinv = lax.rsqrt(jnp.maximum(deg, 1.0))
    dinv_ref[...] = dinv
    xw = jnp.dot(x_ref[...], w_ref[...], preferred_element_type=jnp.float32)
    y1_ref[...] = xw * dinv[:, 0:1]


_tc_a = pl.pallas_call(
    _tc_a_body,
    grid=(_N // _R,),
    in_specs=[
        pl.BlockSpec((_R, _D), lambda i: (i, 0)),
        pl.BlockSpec((_D, _HP1), lambda i: (0, 0)),
        pl.BlockSpec((_NC, _R, _DW), lambda i: (0, i, 0)),
    ],
    out_specs=[
        pl.BlockSpec((_R, _HP1), lambda i: (i, 0)),
        pl.BlockSpec((_R, _DW), lambda i: (i, 0)),
    ],
    out_shape=[
        jax.ShapeDtypeStruct((_N, _HP1), jnp.float32),
        jax.ShapeDtypeStruct((_N, _DW), jnp.float32),
    ],
)


def _tc_b_body(aggp_ref, y1_ref, dinv_ref, b1_ref, w2_ref, y2_ref):
    dinv = dinv_ref[:, 0:1]
    s = (aggp_ref[0] + aggp_ref[1] + y1_ref[...]) * dinv + b1_ref[...]
    h1 = jnp.tanh(s)
    y2_ref[...] = (
        jnp.dot(h1, w2_ref[...], preferred_element_type=jnp.float32) * dinv
    )


_tc_b = pl.pallas_call(
    _tc_b_body,
    grid=(_N // _R,),
    in_specs=[
        pl.BlockSpec((_NC, _R, _HP1), lambda i: (0, i, 0)),
        pl.BlockSpec((_R, _HP1), lambda i: (i, 0)),
        pl.BlockSpec((_R, _DW), lambda i: (i, 0)),
        pl.BlockSpec((1, _HP1), lambda i: (0, 0)),
        pl.BlockSpec((_HP1, _HP2), lambda i: (0, 0)),
    ],
    out_specs=pl.BlockSpec((_R, _HP2), lambda i: (i, 0)),
    out_shape=jax.ShapeDtypeStruct((_N, _HP2), jnp.float32),
)


def _tc_c_body(aggp_ref, y2_ref, dinv_ref, b2_ref, wc_ref, bc_ref,
               h_ref, out_ref):
    dinv = dinv_ref[:, 0:1]
    s = (aggp_ref[0] + aggp_ref[1] + y2_ref[...]) * dinv + b2_ref[...]
    h = jnp.tanh(s)
    h_ref[...] = h[:, 0:_H2]
    out_ref[...] = (
        jnp.dot(h, wc_ref[...], preferred_element_type=jnp.float32)
        + bc_ref[...]
    )


_tc_c = pl.pallas_call(
    _tc_c_body,
    grid=(_N // _R,),
    in_specs=[
        pl.BlockSpec((_NC, _R, _HP2), lambda i: (0, i, 0)),
        pl.BlockSpec((_R, _HP2), lambda i: (i, 0)),
        pl.BlockSpec((_R, _DW), lambda i: (i, 0)),
        pl.BlockSpec((1, _HP2), lambda i: (0, 0)),
        pl.BlockSpec((_HP2, _C), lambda i: (0, 0)),
        pl.BlockSpec((1, _C), lambda i: (0, 0)),
    ],
    out_specs=[
        pl.BlockSpec((_R, _H2), lambda i: (i, 0)),
        pl.BlockSpec((_R, _C), lambda i: (i, 0)),
    ],
    out_shape=[
        jax.ShapeDtypeStruct((_N, _H2), jnp.float32),
        jax.ShapeDtypeStruct((_N, _C), jnp.float32),
    ],
)


def kernel(x, edge_index, W1, b1, W2, b2, Wc, bc):
    npad = _EP - _E
    ar = jnp.arange(npad, dtype=jnp.int32)
    src = jnp.concatenate([edge_index[0], ar % _N]).reshape(_NW, _EC, _EB)
    dst = jnp.concatenate(
        [edge_index[1], _N + 16 + (ar % (_NP - _N - 16))]
    ).reshape(_NW, _EC, _EB)

    ones8 = jnp.ones((_EB, _DW), jnp.float32)
    z8 = jnp.zeros((_ZB, _DW), jnp.float32)
    zrows1 = jnp.zeros((_ZB, _HP1), jnp.float32)
    zrows2 = jnp.zeros((_ZB, _HP2), jnp.float32)

    W1p = jnp.pad(W1, ((0, 0), (0, _HP1 - _H1)))
    b1p = jnp.pad(b1, (0, _HP1 - _H1)).reshape(1, _HP1)
    W2p = jnp.pad(W2, ((0, _HP1 - _H1), (0, _HP2 - _H2)))
    b2p = jnp.pad(b2, (0, _HP2 - _H2)).reshape(1, _HP2)
    Wcp = jnp.pad(Wc, ((0, _HP2 - _H2), (0, 0)))
    bcp = bc.reshape(1, _C)

    degp = _deg(dst, ones8, z8)
    y1, dinv8 = _tc_a(x, W1p, degp)
    agg1 = _agg1(y1, src, dst, zrows1)
    y2 = _tc_b(agg1, y1, dinv8, b1p, W2p)
    agg2 = _agg2(y2, src, dst, zrows2)
    h, out = _tc_c(agg2, y2, dinv8, b2p, Wcp, bcp)
    return (out, h)


# speedup vs baseline: 49.8107x; 1.0337x over previous
import functools

import jax
import jax.numpy as jnp
from jax import lax
from jax.experimental import pallas as pl
from jax.experimental.pallas import tpu as pltpu
from jax.experimental.pallas import tpu_sc as plsc

_N = 10000
_NP = 10240
_E = 320000
_D = 128
_H1 = 50
_HP1 = 64
_H2 = 2
_HP2 = 16
_C = 10
_DW = 8

_NC = 2
_NS = 16
_NW = _NC * _NS
_EP = 327680
_EB = 128
_EC = 80
_RPS = _NP // _NS
_ZB = 32
_ZC = _RPS // _ZB
_NB = 8
_NG = _EC // _NB

_mesh = functools.partial(
    plsc.VectorSubcoreMesh, core_axis_name="c", subcore_axis_name="s"
)


def _deg_body(edge_hbm, ones_hbm, zrows_hbm, out_hbm, dst_v, ones_v, zrows_v,
              acc_sh, sem):
    cid = lax.axis_index("c")
    sid = lax.axis_index("s")
    wid = sid * _NC + cid
    base = sid * _RPS
    pltpu.sync_copy(zrows_hbm, zrows_v)
    for t in range(_ZC):
        pltpu.sync_copy(zrows_v, acc_sh.at[pl.ds(base + t * _ZB, _ZB)])
    pltpu.sync_copy(ones_hbm, ones_v)
    pltpu.sync_copy(edge_hbm.at[1, pl.ds(wid * _EC, _EC)], dst_v)
    plsc.subcore_barrier()

    def group(g, carry):
        descs = []
        for b in range(_NB):
            descs.append(
                pltpu.async_copy(ones_v, acc_sh.at[dst_v.at[g * _NB + b]],
                                 sem.at[b], add=True))
        for d in descs:
            d.wait()
        return carry

    lax.fori_loop(0, _NG, group, 0)
    plsc.subcore_barrier()
    pltpu.sync_copy(acc_sh.at[pl.ds(base, _RPS)],
                    out_hbm.at[cid, pl.ds(base, _RPS)])


_deg = pl.kernel(
    _deg_body,
    out_type=jax.ShapeDtypeStruct((_NC, _NP, _DW), jnp.float32),
    mesh=_mesh(),
    scratch_types=[
        pltpu.VMEM((_EC, _EB), jnp.int32),
        pltpu.VMEM((_EB, _DW), jnp.float32),
        pltpu.VMEM((_ZB, _DW), jnp.float32),
        pltpu.VMEM_SHARED((_NP, _DW), jnp.float32),
        pltpu.SemaphoreType.DMA((_NB,)),
    ],
    compiler_params=pltpu.CompilerParams(use_tc_tiling_on_sc=False),
)


def _make_agg(width):
    def body(y_hbm, edge_hbm, zrows_hbm, out_hbm,
             src_v, dst_v, rows_v, zrows_v, acc_sh, sem_g, sem_s):
        cid = lax.axis_index("c")
        sid = lax.axis_index("s")
        wid = sid * _NC + cid
        base = sid * _RPS
        pltpu.sync_copy(zrows_hbm, zrows_v)
        for t in range(_ZC):
            pltpu.sync_copy(zrows_v, acc_sh.at[pl.ds(base + t * _ZB, _ZB)])
        pltpu.sync_copy(edge_hbm.at[0, pl.ds(wid * _EC, _EC)], src_v)
        pltpu.sync_copy(edge_hbm.at[1, pl.ds(wid * _EC, _EC)], dst_v)
        plsc.subcore_barrier()

        def group(g, carry):
            gds = []
            for b in range(_NB):
                j = g * _NB + b

                @pl.when(g > 0)
                def _wait_prev_scatter():
                    pltpu.make_async_copy(
                        rows_v.at[b], acc_sh.at[dst_v.at[j - _NB]],
                        sem_s.at[b]).wait()

                gds.append(
                    pltpu.async_copy(y_hbm.at[src_v.at[j]], rows_v.at[b],
                                     sem_g.at[b]))
            for b in range(_NB):
                j = g * _NB + b
                gds[b].wait()
                pltpu.async_copy(rows_v.at[b], acc_sh.at[dst_v.at[j]],
                                 sem_s.at[b], add=True)
            return carry

        lax.fori_loop(0, _NG, group, 0)
        for b in range(_NB):
            j = (_NG - 1) * _NB + b
            pltpu.make_async_copy(
                rows_v.at[b], acc_sh.at[dst_v.at[j]], sem_s.at[b]).wait()
        plsc.subcore_barrier()
        pltpu.sync_copy(acc_sh.at[pl.ds(base, _RPS)],
                        out_hbm.at[cid, pl.ds(base, _RPS)])

    return pl.kernel(
        body,
        out_type=jax.ShapeDtypeStruct((_NC, _NP, width), jnp.float32),
        mesh=_mesh(),
        scratch_types=[
            pltpu.VMEM((_EC, _EB), jnp.int32),
            pltpu.VMEM((_EC, _EB), jnp.int32),
            pltpu.VMEM((_NB, _EB, width), jnp.float32),
            pltpu.VMEM((_ZB, width), jnp.float32),
            pltpu.VMEM_SHARED((_NP, width), jnp.float32),
            pltpu.SemaphoreType.DMA((_NB,)),
            pltpu.SemaphoreType.DMA((_NB,)),
        ],
        compiler_params=pltpu.CompilerParams(use_tc_tiling_on_sc=False),
    )


_agg1 = _make_agg(_HP1)
_agg2 = _make_agg(_HP2)

_R = 1000


def _tc_a0_body(x_ref, w_ref, xw_ref):
    xw_ref[...] = jnp.dot(x_ref[...], w_ref[...],
                          preferred_element_type=jnp.float32)


_tc_a0 = pl.pallas_call(
    _tc_a0_body,
    grid=(_N // _R,),
    in_specs=[
        pl.BlockSpec((_R, _D), lambda i: (i, 0)),
        pl.BlockSpec((_D, _HP1), lambda i: (0, 0)),
    ],
    out_specs=pl.BlockSpec((_R, _HP1), lambda i: (i, 0)),
    out_shape=jax.ShapeDtypeStruct((_N, _HP1), jnp.float32),
)


def _tc_a1_body(xw_ref, degp_ref, y1_ref, dinv_ref):
    deg = degp_ref[0] + degp_ref[1] + 1.0
    dinv = lax.rsqrt(jnp.maximum(deg, 1.0))
    dinv_ref[...] = dinv
    y1_ref[...] = xw_ref[...] * dinv[:, 0:1]


_tc_a1 = pl.pallas_call(
    _tc_a1_body,
    grid=(_N // _R,),
    in_specs=[
        pl.BlockSpec((_R, _HP1), lambda i: (i, 0)),
        pl.BlockSpec((_NC, _R, _DW), lambda i: (0, i, 0)),
    ],
    out_specs=[
        pl.BlockSpec((_R, _HP1), lambda i: (i, 0)),
        pl.BlockSpec((_R, _DW), lambda i: (i, 0)),
    ],
    out_shape=[
        jax.ShapeDtypeStruct((_N, _HP1), jnp.float32),
        jax.ShapeDtypeStruct((_N, _DW), jnp.float32),
    ],
)


def _tc_b_body(aggp_ref, y1_ref, dinv_ref, b1_ref, w2_ref, y2_ref):
    dinv = dinv_ref[:, 0:1]
    s = (aggp_ref[0] + aggp_ref[1] + y1_ref[...]) * dinv + b1_ref[...]
    h1 = jnp.tanh(s)
    y2_ref[...] = (
        jnp.dot(h1, w2_ref[...], preferred_element_type=jnp.float32) * dinv
    )


_tc_b = pl.pallas_call(
    _tc_b_body,
    grid=(_N // _R,),
    in_specs=[
        pl.BlockSpec((_NC, _R, _HP1), lambda i: (0, i, 0)),
        pl.BlockSpec((_R, _HP1), lambda i: (i, 0)),
        pl.BlockSpec((_R, _DW), lambda i: (i, 0)),
        pl.BlockSpec((1, _HP1), lambda i: (0, 0)),
        pl.BlockSpec((_HP1, _HP2), lambda i: (0, 0)),
    ],
    out_specs=pl.BlockSpec((_R, _HP2), lambda i: (i, 0)),
    out_shape=jax.ShapeDtypeStruct((_N, _HP2), jnp.float32),
)


def _tc_c_body(aggp_ref, y2_ref, dinv_ref, b2_ref, wc_ref, bc_ref,
               h_ref, out_ref):
    dinv = dinv_ref[:, 0:1]
    s = (aggp_ref[0] + aggp_ref[1] + y2_ref[...]) * dinv + b2_ref[...]
    h = jnp.tanh(s)
    h_ref[...] = h[:, 0:_H2]
    out_ref[...] = (
        jnp.dot(h, wc_ref[...], preferred_element_type=jnp.float32)
        + bc_ref[...]
    )


_tc_c = pl.pallas_call(
    _tc_c_body,
    grid=(_N // _R,),
    in_specs=[
        pl.BlockSpec((_NC, _R, _HP2), lambda i: (0, i, 0)),
        pl.BlockSpec((_R, _HP2), lambda i: (i, 0)),
        pl.BlockSpec((_R, _DW), lambda i: (i, 0)),
        pl.BlockSpec((1, _HP2), lambda i: (0, 0)),
        pl.BlockSpec((_HP2, _C), lambda i: (0, 0)),
        pl.BlockSpec((1, _C), lambda i: (0, 0)),
    ],
    out_specs=[
        pl.BlockSpec((_R, _H2), lambda i: (i, 0)),
        pl.BlockSpec((_R, _C), lambda i: (i, 0)),
    ],
    out_shape=[
        jax.ShapeDtypeStruct((_N, _H2), jnp.float32),
        jax.ShapeDtypeStruct((_N, _C), jnp.float32),
    ],
)


def kernel(x, edge_index, W1, b1, W2, b2, Wc, bc):
    npad = _EP - _E
    ar = jnp.arange(npad, dtype=jnp.int32)
    pads = jnp.stack([ar % _N, _N + 16 + (ar % (_NP - _N - 16))])
    edges = jnp.concatenate([edge_index, pads], axis=1).reshape(
        2, _NW * _EC, _EB)

    ones8 = jnp.ones((_EB, _DW), jnp.float32)
    z8 = jnp.zeros((_ZB, _DW), jnp.float32)
    zrows1 = jnp.zeros((_ZB, _HP1), jnp.float32)
    zrows2 = jnp.zeros((_ZB, _HP2), jnp.float32)

    W1p = jnp.pad(W1, ((0, 0), (0, _HP1 - _H1)))
    b1p = jnp.pad(b1, (0, _HP1 - _H1)).reshape(1, _HP1)
    W2p = jnp.pad(W2, ((0, _HP1 - _H1), (0, _HP2 - _H2)))
    b2p = jnp.pad(b2, (0, _HP2 - _H2)).reshape(1, _HP2)
    Wcp = jnp.pad(Wc, ((0, _HP2 - _H2), (0, 0)))
    bcp = bc.reshape(1, _C)

    degp = _deg(edges, ones8, z8)
    xw = _tc_a0(x, W1p)
    y1, dinv8 = _tc_a1(xw, degp)
    agg1 = _agg1(y1, edges, zrows1)
    y2 = _tc_b(agg1, y1, dinv8, b1p, W2p)
    agg2 = _agg2(y2, edges, zrows2)
    h, out = _tc_c(agg2, y2, dinv8, b2p, Wcp, bcp)
    return (out, h)
